# baseline (speedup 1.0000x reference)
import functools

import jax
import jax.numpy as jnp
from jax import lax
from jax.experimental import pallas as pl
from jax.experimental.pallas import tpu as pltpu
from jax.experimental.pallas import tpu_sc as plsc

N = 16384
D = 256
C = 8
K = 8192
NT = 256
KC = 512
NKC = K // KC


def _c2_kernel(cb_ref, out_ref):
    cb = cb_ref[0]
    c2 = jnp.sum(cb * cb, axis=1)
    out_ref[0] = c2.reshape(NKC, KC)


def _c2_all(codebooks):
    return pl.pallas_call(
        _c2_kernel,
        grid=(C,),
        in_specs=[pl.BlockSpec((1, K, D), lambda i: (i, 0, 0))],
        out_specs=pl.BlockSpec((1, NKC, KC), lambda i: (i, 0, 0)),
        out_shape=jax.ShapeDtypeStruct((C, NKC, KC), jnp.float32),
    )(codebooks)


def _argmin_body(r, cb_ref, c2_ref, idx_ref):
    x2 = jnp.sum(r * r, axis=1, keepdims=True)
    rb = r.astype(jnp.bfloat16)

    def body(kc, carry):
        run_min, run_idx = carry
        cbc = cb_ref[pl.ds(kc * KC, KC), :]
        mm = lax.dot_general(
            rb, cbc.astype(jnp.bfloat16),
            (((1,), (1,)), ((), ())),
            preferred_element_type=jnp.float32,
        )
        c2c = c2_ref[pl.ds(kc, 1), :]
        d = jnp.sqrt(jnp.maximum((x2 + c2c) - 2.0 * mm, 0.0))
        cmin = jnp.min(d, axis=1, keepdims=True)
        io = lax.broadcasted_iota(jnp.int32, (NT, KC), 1)
        carg = jnp.min(
            jnp.where(d == cmin, io, KC), axis=1, keepdims=True
        ) + kc * KC
        take = cmin < run_min
        return (jnp.where(take, cmin, run_min), jnp.where(take, carg, run_idx))

    init = (
        jnp.full((NT, 1), jnp.inf, jnp.float32),
        jnp.zeros((NT, 1), jnp.int32),
    )
    _, run_idx = lax.fori_loop(0, NKC, body, init)
    idx_ref[...] = run_idx


def _stage0_kernel(x_ref, cb_ref, c2_ref, idx_ref):
    _argmin_body(x_ref[...], cb_ref, c2_ref, idx_ref)


def _stage_kernel(r_ref, q_ref, cb_ref, c2_ref, idx_ref, rout_ref):
    r = r_ref[...] - q_ref[...]
    rout_ref[...] = r
    _argmin_body(r, cb_ref, c2_ref, idx_ref)


_ROW_SPEC = pl.BlockSpec((NT, D), lambda n: (n, 0))
_CB_SPEC = pl.BlockSpec((K, D), lambda n: (0, 0))
_C2_SPEC = pl.BlockSpec((NKC, KC), lambda n: (0, 0))
_IDX_SPEC = pl.BlockSpec((NT, 1), lambda n: (n, 0))


def _stage0(x, cb, c2):
    return pl.pallas_call(
        _stage0_kernel,
        grid=(N // NT,),
        in_specs=[_ROW_SPEC, _CB_SPEC, _C2_SPEC],
        out_specs=_IDX_SPEC,
        out_shape=jax.ShapeDtypeStruct((N, 1), jnp.int32),
    )(x, cb, c2)


def _stage(r, q, cb, c2):
    return pl.pallas_call(
        _stage_kernel,
        grid=(N // NT,),
        in_specs=[_ROW_SPEC, _ROW_SPEC, _CB_SPEC, _C2_SPEC],
        out_specs=(_IDX_SPEC, _ROW_SPEC),
        out_shape=(
            jax.ShapeDtypeStruct((N, 1), jnp.int32),
            jax.ShapeDtypeStruct((N, D), jnp.float32),
        ),
    )(r, q, cb, c2)


_NW = 32
_BW = N // _NW
_CH = 128


@functools.cache
def _sc_gather_kernel():
    @functools.partial(
        pl.kernel,
        mesh=plsc.VectorSubcoreMesh(core_axis_name="c", subcore_axis_name="s"),
        out_type=jax.ShapeDtypeStruct((N, D), jnp.float32),
        scratch_types=[
            pltpu.VMEM((_BW,), jnp.int32),
            pltpu.VMEM((_CH, D), jnp.float32),
            pltpu.VMEM((_CH, D), jnp.float32),
            pltpu.SemaphoreType.DMA,
            pltpu.SemaphoreType.DMA,
        ],
    )
    def gather(cb_hbm, idx_hbm, q_hbm, idx_v, rows_a, rows_b, sem_a, sem_b):
        wid = lax.axis_index("s") * 2 + lax.axis_index("c")
        base = wid * _BW
        pltpu.sync_copy(idx_hbm.at[pl.ds(base, _BW)], idx_v)
        bufs = ((rows_a, sem_a), (rows_b, sem_b))
        copies = []
        for ch in range(_BW // _CH):
            rows, sem = bufs[ch % 2]
            copies.append(
                pltpu.async_copy(cb_hbm.at[idx_v.at[pl.ds(ch * _CH, _CH)]], rows, sem)
            )
            if ch >= 1:
                prows, _ = bufs[(ch - 1) % 2]
                copies[ch - 1].wait()
                pltpu.sync_copy(prows, q_hbm.at[pl.ds(base + (ch - 1) * _CH, _CH)])
        last = _BW // _CH - 1
        lrows, _ = bufs[last % 2]
        copies[last].wait()
        pltpu.sync_copy(lrows, q_hbm.at[pl.ds(base + last * _CH, _CH)])

    return gather


def _sc_gather(cb, idx):
    return _sc_gather_kernel()(cb, idx)


def _final_kernel(x_ref, r_ref, q_ref, out_ref):
    out_ref[...] = x_ref[...] - (r_ref[...] - q_ref[...])


def _final(x, r, q):
    return pl.pallas_call(
        _final_kernel,
        grid=(N // NT,),
        in_specs=[_ROW_SPEC, _ROW_SPEC, _ROW_SPEC],
        out_specs=_ROW_SPEC,
        out_shape=jax.ShapeDtypeStruct((N, D), jnp.float32),
    )(x, r, q)


def kernel(x, codebooks):
    c2 = _c2_all(codebooks)
    idxs = []
    r = x
    q = None
    for i in range(C):
        cb = codebooks[i]
        if i == 0:
            idx2d = _stage0(x, cb, c2[i])
        else:
            idx2d, r = _stage(r, q, cb, c2[i])
        idx = idx2d.reshape(N)
        q = _sc_gather(cb, idx)
        idxs.append(idx)
    out = _final(x, r, q)
    return out, jnp.stack(idxs, axis=1)
